# baseline (speedup 1.0000x reference)
import functools

import jax
import jax.numpy as jnp
from jax import lax
from jax.experimental import pallas as pl
from jax.experimental.pallas import tpu as pltpu
from jax.experimental.pallas import tpu_sc as plsc

N = 10000
F_IN = 256
HID = 512
CLS = 64
E = 160000

NC = 2
NS = 16
LANE = 128
CHUNK = 96

ACC_ROWS = 10016
ZROWS = 626
OROWS = 624
TAIL = N - NS * OROWS

EPAD = 161280
CH_AGG = 105
EPAD_DEG = 163840
CH_DEG = 40

ROW_TILE = 1000
GRID_TC = N // ROW_TILE


IDXB = 15


def _agg_body(nbuf, nchunks, src_hbm, dst_hbm, table_hbm, zeros_hbm, out_hbm,
              src_v, dst_v, rows_v, acc, gsem, ssem, isem):
    c = lax.axis_index("c")
    s = lax.axis_index("s")
    nblocks = nchunks // IDXB
    pltpu.sync_copy(src_hbm.at[c, s, pl.ds(0, IDXB)],
                    src_v.at[pl.ds(0, IDXB)])
    pltpu.sync_copy(dst_hbm.at[s, pl.ds(0, IDXB)], dst_v.at[pl.ds(0, IDXB)])
    if nblocks > 1:
        pltpu.async_copy(src_hbm.at[c, s, pl.ds(IDXB, IDXB)],
                         src_v.at[pl.ds(IDXB, IDXB)], isem)
        pltpu.async_copy(dst_hbm.at[s, pl.ds(IDXB, IDXB)],
                         dst_v.at[pl.ds(IDXB, IDXB)], isem)
    for k in range(min(nbuf - 1, nchunks)):
        pltpu.async_copy(table_hbm.at[src_v.at[k]], rows_v.at[k],
                         gsem.at[k])
    pltpu.sync_copy(zeros_hbm, acc.at[pl.ds(s * ZROWS, ZROWS)])
    plsc.subcore_barrier()

    def chunk(j, carry):
        b = lax.rem(j, nbuf)
        pf = j + nbuf - 1
        nb = lax.rem(pf, nbuf)
        g = j // IDXB
        jj = j - g * IDXB
        row = lax.rem(g, 2) * IDXB + jj

        @pl.when(jnp.logical_and(jj == IDXB - 4, g + 1 < nblocks))
        def _idx_wait():
            pltpu.make_async_copy(src_hbm.at[c, s, pl.ds(0, IDXB)],
                                  src_v.at[pl.ds(0, IDXB)], isem).wait()
            pltpu.make_async_copy(dst_hbm.at[s, pl.ds(0, IDXB)],
                                  dst_v.at[pl.ds(0, IDXB)], isem).wait()

        @pl.when(pf < nchunks)
        def _prefetch():
            gp = pf // IDXB
            rowp = lax.rem(gp, 2) * IDXB + (pf - gp * IDXB)

            @pl.when(j >= 1)
            def _buf_free():
                pltpu.make_async_copy(rows_v.at[nb], acc.at[pl.ds(0, CHUNK)],
                                      ssem.at[nb]).wait()
            pltpu.async_copy(table_hbm.at[src_v.at[rowp]], rows_v.at[nb],
                             gsem.at[nb])

        pltpu.make_async_copy(table_hbm.at[src_v.at[row]], rows_v.at[b],
                              gsem.at[b]).wait()
        pltpu.async_copy(rows_v.at[b], acc.at[dst_v.at[row]], ssem.at[b],
                         add=True)

        @pl.when(jnp.logical_and(jj == 0, jnp.logical_and(g >= 1,
                                                          g + 1 < nblocks)))
        def _idx_prefetch():
            noff = lax.rem(g + 1, 2) * IDXB
            pltpu.async_copy(src_hbm.at[c, s, pl.ds((g + 1) * IDXB, IDXB)],
                             src_v.at[pl.ds(noff, IDXB)], isem)
            pltpu.async_copy(dst_hbm.at[s, pl.ds((g + 1) * IDXB, IDXB)],
                             dst_v.at[pl.ds(noff, IDXB)], isem)
        return carry

    lax.fori_loop(0, nchunks, chunk, 0)
    for k in range(min(nbuf, nchunks)):
        pltpu.make_async_copy(rows_v.at[k], acc.at[pl.ds(0, CHUNK)],
                              ssem.at[k]).wait()
    plsc.subcore_barrier()
    pltpu.sync_copy(acc.at[pl.ds(s * OROWS, OROWS)],
                    out_hbm.at[c, pl.ds(s * OROWS, OROWS)])

    @pl.when(s == 0)
    def _copy_tail():
        pltpu.sync_copy(acc.at[pl.ds(NS * OROWS, TAIL)],
                        out_hbm.at[c, pl.ds(NS * OROWS, TAIL)])


def _make_agg(feat, nchunks, nbuf):
    mesh = plsc.VectorSubcoreMesh(core_axis_name="c", subcore_axis_name="s")
    return pl.kernel(
        functools.partial(_agg_body, nbuf, nchunks),
        out_type=jax.ShapeDtypeStruct((NC, N, feat), jnp.float32),
        mesh=mesh,
        scratch_types=[
            pltpu.VMEM((2 * IDXB, CHUNK), jnp.int32),
            pltpu.VMEM((2 * IDXB, CHUNK), jnp.int32),
            pltpu.VMEM((nbuf, CHUNK, feat), jnp.float32),
            pltpu.VMEM_SHARED((ACC_ROWS, feat), jnp.float32),
            pltpu.SemaphoreType.DMA((nbuf,)),
            pltpu.SemaphoreType.DMA((nbuf,)),
            pltpu.SemaphoreType.DMA,
        ],
        compiler_params=pltpu.CompilerParams(use_tc_tiling_on_sc=False),
    )


def _deg_body(dst_hbm, ones_hbm, zeros_hbm, out_hbm, dst_v, ones_v, acc,
              ssem):
    c = lax.axis_index("c")
    s = lax.axis_index("s")
    pltpu.sync_copy(dst_hbm.at[c, s], dst_v)
    pltpu.sync_copy(ones_hbm, ones_v)
    pltpu.sync_copy(zeros_hbm, acc.at[pl.ds(s * ZROWS, ZROWS)])
    plsc.subcore_barrier()

    def chunk(j, carry):
        pltpu.async_copy(ones_v, acc.at[dst_v.at[j]], ssem, add=True)
        return carry

    lax.fori_loop(0, CH_DEG, chunk, 0)

    def drain(j, carry):
        pltpu.make_async_copy(ones_v, acc.at[pl.ds(0, LANE)], ssem).wait()
        return carry

    lax.fori_loop(0, CH_DEG, drain, 0)
    plsc.subcore_barrier()
    pltpu.sync_copy(acc.at[pl.ds(s * OROWS, OROWS)],
                    out_hbm.at[c, pl.ds(s * OROWS, OROWS)])

    @pl.when(s == 0)
    def _copy_tail():
        pltpu.sync_copy(acc.at[pl.ds(NS * OROWS, TAIL)],
                        out_hbm.at[c, pl.ds(NS * OROWS, TAIL)])


def _make_deg():
    mesh = plsc.VectorSubcoreMesh(core_axis_name="c", subcore_axis_name="s")
    return pl.kernel(
        _deg_body,
        out_type=jax.ShapeDtypeStruct((NC, N, 8), jnp.float32),
        mesh=mesh,
        scratch_types=[
            pltpu.VMEM((CH_DEG, LANE), jnp.int32),
            pltpu.VMEM((LANE, 8), jnp.float32),
            pltpu.VMEM_SHARED((ACC_ROWS, 8), jnp.float32),
            pltpu.SemaphoreType.DMA,
        ],
        compiler_params=pltpu.CompilerParams(use_tc_tiling_on_sc=False),
    )


def _scale_body(x_ref, d0_ref, d1_ref, xs_ref, dinv_ref, deginv_ref):
    deg = d0_ref[...] + d1_ref[...] + 1.0
    dinv = lax.rsqrt(deg)
    deginv = 1.0 / deg
    dinv_ref[...] = jnp.concatenate([dinv, dinv], axis=1)
    deginv_ref[...] = jnp.concatenate([deginv, deginv], axis=1)
    xs = x_ref[...] * dinv[:, :1]
    xs_ref[0] = xs[:, :128]
    xs_ref[1] = xs[:, 128:]


def _scale_call(x, d0, d1):
    return pl.pallas_call(
        _scale_body,
        grid=(GRID_TC,),
        in_specs=[
            pl.BlockSpec((ROW_TILE, F_IN), lambda i: (i, 0)),
            pl.BlockSpec((ROW_TILE, 8), lambda i: (i, 0)),
            pl.BlockSpec((ROW_TILE, 8), lambda i: (i, 0)),
        ],
        out_specs=[
            pl.BlockSpec((NC, ROW_TILE, 128), lambda i: (0, i, 0)),
            pl.BlockSpec((ROW_TILE, 16), lambda i: (i, 0)),
            pl.BlockSpec((ROW_TILE, 16), lambda i: (i, 0)),
        ],
        out_shape=[
            jax.ShapeDtypeStruct((NC, N, 128), jnp.float32),
            jax.ShapeDtypeStruct((N, 16), jnp.float32),
            jax.ShapeDtypeStruct((N, 16), jnp.float32),
        ],
    )(x, d0, d1)


def _dense_body(agg_ref, x_ref, dinv_ref, deginv_ref, w1_ref, b1_ref, w2_ref,
                z_ref, zs_ref):
    dinv = dinv_ref[:, :1]
    agg = jnp.concatenate([agg_ref[0], agg_ref[1]], axis=1)
    pre = dinv * agg + deginv_ref[:, :1] * x_ref[...]
    h = jnp.dot(pre, w1_ref[...], preferred_element_type=jnp.float32)
    h = jnp.maximum(h + b1_ref[...], 0.0)
    z = jnp.dot(h, w2_ref[...], preferred_element_type=jnp.float32)
    z_ref[...] = z
    zs = dinv * z
    zs_ref[0] = zs[:, :32]
    zs_ref[1] = zs[:, 32:]


def _dense_call(agg1, x, dinv, deginv, W1, b1, W2):
    return pl.pallas_call(
        _dense_body,
        grid=(GRID_TC,),
        in_specs=[
            pl.BlockSpec((NC, ROW_TILE, 128), lambda i: (0, i, 0)),
            pl.BlockSpec((ROW_TILE, F_IN), lambda i: (i, 0)),
            pl.BlockSpec((ROW_TILE, 16), lambda i: (i, 0)),
            pl.BlockSpec((ROW_TILE, 16), lambda i: (i, 0)),
            pl.BlockSpec((F_IN, HID), lambda i: (0, 0)),
            pl.BlockSpec((1, HID), lambda i: (0, 0)),
            pl.BlockSpec((HID, CLS), lambda i: (0, 0)),
        ],
        out_specs=[
            pl.BlockSpec((ROW_TILE, CLS), lambda i: (i, 0)),
            pl.BlockSpec((NC, ROW_TILE, 32), lambda i: (0, i, 0)),
        ],
        out_shape=[
            jax.ShapeDtypeStruct((N, CLS), jnp.float32),
            jax.ShapeDtypeStruct((NC, N, 32), jnp.float32),
        ],
    )(agg1, x, dinv, deginv, W1, b1, W2)


def _finish_body(agg_ref, z_ref, dinv_ref, deginv_ref, b2_ref, out_ref):
    agg = jnp.concatenate([agg_ref[0], agg_ref[1]], axis=1)
    out_ref[...] = (dinv_ref[:, :1] * agg + deginv_ref[:, :1] * z_ref[...]
                    + b2_ref[...])


def _finish_call(agg2, z, dinv, deginv, b2):
    return pl.pallas_call(
        _finish_body,
        grid=(GRID_TC,),
        in_specs=[
            pl.BlockSpec((NC, ROW_TILE, 32), lambda i: (0, i, 0)),
            pl.BlockSpec((ROW_TILE, CLS), lambda i: (i, 0)),
            pl.BlockSpec((ROW_TILE, 16), lambda i: (i, 0)),
            pl.BlockSpec((ROW_TILE, 16), lambda i: (i, 0)),
            pl.BlockSpec((1, CLS), lambda i: (0, 0)),
        ],
        out_specs=pl.BlockSpec((ROW_TILE, CLS), lambda i: (i, 0)),
        out_shape=jax.ShapeDtypeStruct((N, CLS), jnp.float32),
    )(agg2, z, dinv, deginv, b2)


def kernel(x, edge_index, W1, b1, W2, b2):
    src = edge_index[0]
    dst = edge_index[1]

    pad1 = EPAD - E
    src_p = jnp.concatenate([src, jnp.zeros((pad1,), jnp.int32)])
    src4 = jnp.stack([src_p, src_p + N]).reshape(NC, NS, CH_AGG, CHUNK)
    dst4 = jnp.concatenate(
        [dst, jnp.full((pad1,), N, jnp.int32)]).reshape(NS, CH_AGG, CHUNK)
    padd = EPAD_DEG - E
    dstd = jnp.concatenate(
        [dst, jnp.full((padd,), N, jnp.int32)]).reshape(NC, NS, CH_DEG, LANE)

    zeros128 = jnp.zeros((ZROWS, 128), jnp.float32)
    zeros32 = jnp.zeros((ZROWS, 32), jnp.float32)
    zeros8 = jnp.zeros((ZROWS, 8), jnp.float32)
    ones8 = jnp.ones((LANE, 8), jnp.float32)

    degp = _make_deg()(dstd, ones8, zeros8)

    xs2, dinv, deginv = _scale_call(x, degp[0], degp[1])

    agg1 = _make_agg(128, CH_AGG, 3)(src4, dst4, xs2.reshape(NC * N, 128),
                                     zeros128)

    z, zs2 = _dense_call(agg1, x, dinv, deginv, W1, b1.reshape(1, HID), W2)

    agg2 = _make_agg(32, CH_AGG, 4)(src4, dst4, zs2.reshape(NC * N, 32),
                                    zeros32)

    return _finish_call(agg2, z, dinv, deginv, b2.reshape(1, CLS))


# speedup vs baseline: 1.0128x; 1.0128x over previous
import functools

import jax
import jax.numpy as jnp
from jax import lax
from jax.experimental import pallas as pl
from jax.experimental.pallas import tpu as pltpu
from jax.experimental.pallas import tpu_sc as plsc

N = 10000
F_IN = 256
HID = 512
CLS = 64
E = 160000

NC = 2
NS = 16
LANE = 128
CHUNK = 96

ACC_ROWS = 10016
ZROWS = 626
OROWS = 624
TAIL = N - NS * OROWS

EPAD = 161280
CH_AGG = 105
EPAD_DEG = 163840
CH_DEG = 40

ROW_TILE = 1000
GRID_TC = N // ROW_TILE


IDXB = 15


def _agg_body(nbuf, nchunks, src_hbm, dst_hbm, table_hbm, zeros_hbm, out_hbm,
              src_v, dst_v, rows_v, acc, gsem, ssem, isem):
    c = lax.axis_index("c")
    s = lax.axis_index("s")
    nblocks = nchunks // IDXB
    pltpu.sync_copy(src_hbm.at[c, s, pl.ds(0, IDXB)],
                    src_v.at[pl.ds(0, IDXB)])
    pltpu.sync_copy(dst_hbm.at[s, pl.ds(0, IDXB)], dst_v.at[pl.ds(0, IDXB)])
    if nblocks > 1:
        pltpu.async_copy(src_hbm.at[c, s, pl.ds(IDXB, IDXB)],
                         src_v.at[pl.ds(IDXB, IDXB)], isem)
        pltpu.async_copy(dst_hbm.at[s, pl.ds(IDXB, IDXB)],
                         dst_v.at[pl.ds(IDXB, IDXB)], isem)
    for k in range(min(nbuf - 1, nchunks)):
        pltpu.async_copy(table_hbm.at[src_v.at[k]], rows_v.at[k],
                         gsem.at[k])
    pltpu.sync_copy(zeros_hbm, acc.at[pl.ds(s * ZROWS, ZROWS)])
    plsc.subcore_barrier()

    def chunk(j, carry):
        b = lax.rem(j, nbuf)
        pf = j + nbuf - 1
        nb = lax.rem(pf, nbuf)
        g = j // IDXB
        jj = j - g * IDXB
        row = lax.rem(g, 2) * IDXB + jj

        @pl.when(jnp.logical_and(jj == IDXB - nbuf, g + 1 < nblocks))
        def _idx_wait():
            pltpu.make_async_copy(src_hbm.at[c, s, pl.ds(0, IDXB)],
                                  src_v.at[pl.ds(0, IDXB)], isem).wait()
            pltpu.make_async_copy(dst_hbm.at[s, pl.ds(0, IDXB)],
                                  dst_v.at[pl.ds(0, IDXB)], isem).wait()

        @pl.when(pf < nchunks)
        def _prefetch():
            gp = pf // IDXB
            rowp = lax.rem(gp, 2) * IDXB + (pf - gp * IDXB)

            @pl.when(j >= 1)
            def _buf_free():
                pltpu.make_async_copy(rows_v.at[nb], acc.at[pl.ds(0, CHUNK)],
                                      ssem.at[nb]).wait()
            pltpu.async_copy(table_hbm.at[src_v.at[rowp]], rows_v.at[nb],
                             gsem.at[nb])

        pltpu.make_async_copy(table_hbm.at[src_v.at[row]], rows_v.at[b],
                              gsem.at[b]).wait()
        pltpu.async_copy(rows_v.at[b], acc.at[dst_v.at[row]], ssem.at[b],
                         add=True)

        @pl.when(jnp.logical_and(jj == 0, jnp.logical_and(g >= 1,
                                                          g + 1 < nblocks)))
        def _idx_prefetch():
            noff = lax.rem(g + 1, 2) * IDXB
            pltpu.async_copy(src_hbm.at[c, s, pl.ds((g + 1) * IDXB, IDXB)],
                             src_v.at[pl.ds(noff, IDXB)], isem)
            pltpu.async_copy(dst_hbm.at[s, pl.ds((g + 1) * IDXB, IDXB)],
                             dst_v.at[pl.ds(noff, IDXB)], isem)
        return carry

    lax.fori_loop(0, nchunks, chunk, 0)
    for k in range(min(nbuf, nchunks)):
        pltpu.make_async_copy(rows_v.at[k], acc.at[pl.ds(0, CHUNK)],
                              ssem.at[k]).wait()
    plsc.subcore_barrier()
    pltpu.sync_copy(acc.at[pl.ds(s * OROWS, OROWS)],
                    out_hbm.at[c, pl.ds(s * OROWS, OROWS)])

    @pl.when(s == 0)
    def _copy_tail():
        pltpu.sync_copy(acc.at[pl.ds(NS * OROWS, TAIL)],
                        out_hbm.at[c, pl.ds(NS * OROWS, TAIL)])


def _make_agg(feat, nchunks, nbuf):
    mesh = plsc.VectorSubcoreMesh(core_axis_name="c", subcore_axis_name="s")
    return pl.kernel(
        functools.partial(_agg_body, nbuf, nchunks),
        out_type=jax.ShapeDtypeStruct((NC, N, feat), jnp.float32),
        mesh=mesh,
        scratch_types=[
            pltpu.VMEM((2 * IDXB, CHUNK), jnp.int32),
            pltpu.VMEM((2 * IDXB, CHUNK), jnp.int32),
            pltpu.VMEM((nbuf, CHUNK, feat), jnp.float32),
            pltpu.VMEM_SHARED((ACC_ROWS, feat), jnp.float32),
            pltpu.SemaphoreType.DMA((nbuf,)),
            pltpu.SemaphoreType.DMA((nbuf,)),
            pltpu.SemaphoreType.DMA,
        ],
        compiler_params=pltpu.CompilerParams(use_tc_tiling_on_sc=False),
    )


def _deg_body(dst_hbm, ones_hbm, zeros_hbm, out_hbm, dst_v, ones_v, acc,
              ssem):
    c = lax.axis_index("c")
    s = lax.axis_index("s")
    pltpu.sync_copy(dst_hbm.at[c, s], dst_v)
    pltpu.sync_copy(ones_hbm, ones_v)
    pltpu.sync_copy(zeros_hbm, acc.at[pl.ds(s * ZROWS, ZROWS)])
    plsc.subcore_barrier()

    def chunk(j, carry):
        pltpu.async_copy(ones_v, acc.at[dst_v.at[j]], ssem, add=True)
        return carry

    lax.fori_loop(0, CH_DEG, chunk, 0)

    def drain(j, carry):
        pltpu.make_async_copy(ones_v, acc.at[pl.ds(0, LANE)], ssem).wait()
        return carry

    lax.fori_loop(0, CH_DEG, drain, 0)
    plsc.subcore_barrier()
    pltpu.sync_copy(acc.at[pl.ds(s * OROWS, OROWS)],
                    out_hbm.at[c, pl.ds(s * OROWS, OROWS)])

    @pl.when(s == 0)
    def _copy_tail():
        pltpu.sync_copy(acc.at[pl.ds(NS * OROWS, TAIL)],
                        out_hbm.at[c, pl.ds(NS * OROWS, TAIL)])


def _make_deg():
    mesh = plsc.VectorSubcoreMesh(core_axis_name="c", subcore_axis_name="s")
    return pl.kernel(
        _deg_body,
        out_type=jax.ShapeDtypeStruct((NC, N, 8), jnp.float32),
        mesh=mesh,
        scratch_types=[
            pltpu.VMEM((CH_DEG, LANE), jnp.int32),
            pltpu.VMEM((LANE, 8), jnp.float32),
            pltpu.VMEM_SHARED((ACC_ROWS, 8), jnp.float32),
            pltpu.SemaphoreType.DMA,
        ],
        compiler_params=pltpu.CompilerParams(use_tc_tiling_on_sc=False),
    )


def _scale_body(x_ref, d0_ref, d1_ref, xs_ref, dinv_ref, deginv_ref):
    deg = d0_ref[...] + d1_ref[...] + 1.0
    dinv = lax.rsqrt(deg)
    deginv = 1.0 / deg
    dinv_ref[...] = jnp.concatenate([dinv, dinv], axis=1)
    deginv_ref[...] = jnp.concatenate([deginv, deginv], axis=1)
    xs = x_ref[...] * dinv[:, :1]
    xs_ref[0] = xs[:, :128]
    xs_ref[1] = xs[:, 128:]


def _scale_call(x, d0, d1):
    return pl.pallas_call(
        _scale_body,
        grid=(GRID_TC,),
        in_specs=[
            pl.BlockSpec((ROW_TILE, F_IN), lambda i: (i, 0)),
            pl.BlockSpec((ROW_TILE, 8), lambda i: (i, 0)),
            pl.BlockSpec((ROW_TILE, 8), lambda i: (i, 0)),
        ],
        out_specs=[
            pl.BlockSpec((NC, ROW_TILE, 128), lambda i: (0, i, 0)),
            pl.BlockSpec((ROW_TILE, 16), lambda i: (i, 0)),
            pl.BlockSpec((ROW_TILE, 16), lambda i: (i, 0)),
        ],
        out_shape=[
            jax.ShapeDtypeStruct((NC, N, 128), jnp.float32),
            jax.ShapeDtypeStruct((N, 16), jnp.float32),
            jax.ShapeDtypeStruct((N, 16), jnp.float32),
        ],
    )(x, d0, d1)


def _dense_body(agg_ref, x_ref, dinv_ref, deginv_ref, w1_ref, b1_ref, w2_ref,
                z_ref, zs_ref):
    dinv = dinv_ref[:, :1]
    agg = jnp.concatenate([agg_ref[0], agg_ref[1]], axis=1)
    pre = dinv * agg + deginv_ref[:, :1] * x_ref[...]
    h = jnp.dot(pre, w1_ref[...], preferred_element_type=jnp.float32)
    h = jnp.maximum(h + b1_ref[...], 0.0)
    z = jnp.dot(h, w2_ref[...], preferred_element_type=jnp.float32)
    z_ref[...] = z
    zs = dinv * z
    zs_ref[0] = zs[:, :32]
    zs_ref[1] = zs[:, 32:]


def _dense_call(agg1, x, dinv, deginv, W1, b1, W2):
    return pl.pallas_call(
        _dense_body,
        grid=(GRID_TC,),
        in_specs=[
            pl.BlockSpec((NC, ROW_TILE, 128), lambda i: (0, i, 0)),
            pl.BlockSpec((ROW_TILE, F_IN), lambda i: (i, 0)),
            pl.BlockSpec((ROW_TILE, 16), lambda i: (i, 0)),
            pl.BlockSpec((ROW_TILE, 16), lambda i: (i, 0)),
            pl.BlockSpec((F_IN, HID), lambda i: (0, 0)),
            pl.BlockSpec((1, HID), lambda i: (0, 0)),
            pl.BlockSpec((HID, CLS), lambda i: (0, 0)),
        ],
        out_specs=[
            pl.BlockSpec((ROW_TILE, CLS), lambda i: (i, 0)),
            pl.BlockSpec((NC, ROW_TILE, 32), lambda i: (0, i, 0)),
        ],
        out_shape=[
            jax.ShapeDtypeStruct((N, CLS), jnp.float32),
            jax.ShapeDtypeStruct((NC, N, 32), jnp.float32),
        ],
    )(agg1, x, dinv, deginv, W1, b1, W2)


def _finish_body(agg_ref, z_ref, dinv_ref, deginv_ref, b2_ref, out_ref):
    agg = jnp.concatenate([agg_ref[0], agg_ref[1]], axis=1)
    out_ref[...] = (dinv_ref[:, :1] * agg + deginv_ref[:, :1] * z_ref[...]
                    + b2_ref[...])


def _finish_call(agg2, z, dinv, deginv, b2):
    return pl.pallas_call(
        _finish_body,
        grid=(GRID_TC,),
        in_specs=[
            pl.BlockSpec((NC, ROW_TILE, 32), lambda i: (0, i, 0)),
            pl.BlockSpec((ROW_TILE, CLS), lambda i: (i, 0)),
            pl.BlockSpec((ROW_TILE, 16), lambda i: (i, 0)),
            pl.BlockSpec((ROW_TILE, 16), lambda i: (i, 0)),
            pl.BlockSpec((1, CLS), lambda i: (0, 0)),
        ],
        out_specs=pl.BlockSpec((ROW_TILE, CLS), lambda i: (i, 0)),
        out_shape=jax.ShapeDtypeStruct((N, CLS), jnp.float32),
    )(agg2, z, dinv, deginv, b2)


def kernel(x, edge_index, W1, b1, W2, b2):
    src = edge_index[0]
    dst = edge_index[1]

    pad1 = EPAD - E
    src_p = jnp.concatenate([src, jnp.zeros((pad1,), jnp.int32)])
    src4 = jnp.stack([src_p, src_p + N]).reshape(NC, NS, CH_AGG, CHUNK)
    dst4 = jnp.concatenate(
        [dst, jnp.full((pad1,), N, jnp.int32)]).reshape(NS, CH_AGG, CHUNK)
    padd = EPAD_DEG - E
    dstd = jnp.concatenate(
        [dst, jnp.full((padd,), N, jnp.int32)]).reshape(NC, NS, CH_DEG, LANE)

    zeros128 = jnp.zeros((ZROWS, 128), jnp.float32)
    zeros32 = jnp.zeros((ZROWS, 32), jnp.float32)
    zeros8 = jnp.zeros((ZROWS, 8), jnp.float32)
    ones8 = jnp.ones((LANE, 8), jnp.float32)

    degp = _make_deg()(dstd, ones8, zeros8)

    xs2, dinv, deginv = _scale_call(x, degp[0], degp[1])

    agg1 = _make_agg(128, CH_AGG, 3)(src4, dst4, xs2.reshape(NC * N, 128),
                                     zeros128)

    z, zs2 = _dense_call(agg1, x, dinv, deginv, W1, b1.reshape(1, HID), W2)

    agg2 = _make_agg(32, CH_AGG, 6)(src4, dst4, zs2.reshape(NC * N, 32),
                                    zeros32)

    return _finish_call(agg2, z, dinv, deginv, b2.reshape(1, CLS))


# speedup vs baseline: 1.0275x; 1.0145x over previous
import functools

import jax
import jax.numpy as jnp
from jax import lax
from jax.experimental import pallas as pl
from jax.experimental.pallas import tpu as pltpu
from jax.experimental.pallas import tpu_sc as plsc

N = 10000
F_IN = 256
HID = 512
CLS = 64
E = 160000

NC = 2
NS = 16
LANE = 128
CHUNK = 96

ACC_ROWS = 10016
ZROWS = 626
OROWS = 624
TAIL = N - NS * OROWS

EPAD = 161280
CH_AGG = 105
EPAD_DEG = 163840
CH_DEG = 40

ROW_TILE = 2000
GRID_TC = N // ROW_TILE


IDXB = 15


def _agg_body(nbuf, nchunks, src_hbm, dst_hbm, table_hbm, zeros_hbm, out_hbm,
              src_v, dst_v, rows_v, acc, gsem, ssem, isem):
    c = lax.axis_index("c")
    s = lax.axis_index("s")
    nblocks = nchunks // IDXB
    pltpu.sync_copy(src_hbm.at[c, s, pl.ds(0, IDXB)],
                    src_v.at[pl.ds(0, IDXB)])
    pltpu.sync_copy(dst_hbm.at[s, pl.ds(0, IDXB)], dst_v.at[pl.ds(0, IDXB)])
    if nblocks > 1:
        pltpu.async_copy(src_hbm.at[c, s, pl.ds(IDXB, IDXB)],
                         src_v.at[pl.ds(IDXB, IDXB)], isem)
        pltpu.async_copy(dst_hbm.at[s, pl.ds(IDXB, IDXB)],
                         dst_v.at[pl.ds(IDXB, IDXB)], isem)
    for k in range(min(nbuf - 1, nchunks)):
        pltpu.async_copy(table_hbm.at[src_v.at[k]], rows_v.at[k],
                         gsem.at[k])
    pltpu.sync_copy(zeros_hbm, acc.at[pl.ds(s * ZROWS, ZROWS)])
    plsc.subcore_barrier()

    def chunk(j, carry):
        b = lax.rem(j, nbuf)
        pf = j + nbuf - 1
        nb = lax.rem(pf, nbuf)
        g = j // IDXB
        jj = j - g * IDXB
        row = lax.rem(g, 2) * IDXB + jj

        @pl.when(jnp.logical_and(jj == IDXB - nbuf, g + 1 < nblocks))
        def _idx_wait():
            pltpu.make_async_copy(src_hbm.at[c, s, pl.ds(0, IDXB)],
                                  src_v.at[pl.ds(0, IDXB)], isem).wait()
            pltpu.make_async_copy(dst_hbm.at[s, pl.ds(0, IDXB)],
                                  dst_v.at[pl.ds(0, IDXB)], isem).wait()

        @pl.when(pf < nchunks)
        def _prefetch():
            gp = pf // IDXB
            rowp = lax.rem(gp, 2) * IDXB + (pf - gp * IDXB)

            @pl.when(j >= 1)
            def _buf_free():
                pltpu.make_async_copy(rows_v.at[nb], acc.at[pl.ds(0, CHUNK)],
                                      ssem.at[nb]).wait()
            pltpu.async_copy(table_hbm.at[src_v.at[rowp]], rows_v.at[nb],
                             gsem.at[nb])

        pltpu.make_async_copy(table_hbm.at[src_v.at[row]], rows_v.at[b],
                              gsem.at[b]).wait()
        pltpu.async_copy(rows_v.at[b], acc.at[dst_v.at[row]], ssem.at[b],
                         add=True)

        @pl.when(jnp.logical_and(jj == 0, jnp.logical_and(g >= 1,
                                                          g + 1 < nblocks)))
        def _idx_prefetch():
            noff = lax.rem(g + 1, 2) * IDXB
            pltpu.async_copy(src_hbm.at[c, s, pl.ds((g + 1) * IDXB, IDXB)],
                             src_v.at[pl.ds(noff, IDXB)], isem)
            pltpu.async_copy(dst_hbm.at[s, pl.ds((g + 1) * IDXB, IDXB)],
                             dst_v.at[pl.ds(noff, IDXB)], isem)
        return carry

    lax.fori_loop(0, nchunks, chunk, 0)
    for k in range(min(nbuf, nchunks)):
        pltpu.make_async_copy(rows_v.at[k], acc.at[pl.ds(0, CHUNK)],
                              ssem.at[k]).wait()
    plsc.subcore_barrier()
    pltpu.sync_copy(acc.at[pl.ds(s * OROWS, OROWS)],
                    out_hbm.at[c, pl.ds(s * OROWS, OROWS)])

    @pl.when(s == 0)
    def _copy_tail():
        pltpu.sync_copy(acc.at[pl.ds(NS * OROWS, TAIL)],
                        out_hbm.at[c, pl.ds(NS * OROWS, TAIL)])


def _make_agg(feat, nchunks, nbuf):
    mesh = plsc.VectorSubcoreMesh(core_axis_name="c", subcore_axis_name="s")
    return pl.kernel(
        functools.partial(_agg_body, nbuf, nchunks),
        out_type=jax.ShapeDtypeStruct((NC, N, feat), jnp.float32),
        mesh=mesh,
        scratch_types=[
            pltpu.VMEM((2 * IDXB, CHUNK), jnp.int32),
            pltpu.VMEM((2 * IDXB, CHUNK), jnp.int32),
            pltpu.VMEM((nbuf, CHUNK, feat), jnp.float32),
            pltpu.VMEM_SHARED((ACC_ROWS, feat), jnp.float32),
            pltpu.SemaphoreType.DMA((nbuf,)),
            pltpu.SemaphoreType.DMA((nbuf,)),
            pltpu.SemaphoreType.DMA,
        ],
        compiler_params=pltpu.CompilerParams(use_tc_tiling_on_sc=False),
    )


def _deg_body(dst_hbm, ones_hbm, zeros_hbm, out_hbm, dst_v, ones_v, acc,
              ssem):
    c = lax.axis_index("c")
    s = lax.axis_index("s")
    pltpu.sync_copy(dst_hbm.at[c, s], dst_v)
    pltpu.sync_copy(ones_hbm, ones_v)
    pltpu.sync_copy(zeros_hbm, acc.at[pl.ds(s * ZROWS, ZROWS)])
    plsc.subcore_barrier()

    def chunk(j, carry):
        pltpu.async_copy(ones_v, acc.at[dst_v.at[j]], ssem, add=True)
        return carry

    lax.fori_loop(0, CH_DEG, chunk, 0)

    def drain(j, carry):
        pltpu.make_async_copy(ones_v, acc.at[pl.ds(0, LANE)], ssem).wait()
        return carry

    lax.fori_loop(0, CH_DEG, drain, 0)
    plsc.subcore_barrier()
    pltpu.sync_copy(acc.at[pl.ds(s * OROWS, OROWS)],
                    out_hbm.at[c, pl.ds(s * OROWS, OROWS)])

    @pl.when(s == 0)
    def _copy_tail():
        pltpu.sync_copy(acc.at[pl.ds(NS * OROWS, TAIL)],
                        out_hbm.at[c, pl.ds(NS * OROWS, TAIL)])


def _make_deg():
    mesh = plsc.VectorSubcoreMesh(core_axis_name="c", subcore_axis_name="s")
    return pl.kernel(
        _deg_body,
        out_type=jax.ShapeDtypeStruct((NC, N, 8), jnp.float32),
        mesh=mesh,
        scratch_types=[
            pltpu.VMEM((CH_DEG, LANE), jnp.int32),
            pltpu.VMEM((LANE, 8), jnp.float32),
            pltpu.VMEM_SHARED((ACC_ROWS, 8), jnp.float32),
            pltpu.SemaphoreType.DMA,
        ],
        compiler_params=pltpu.CompilerParams(use_tc_tiling_on_sc=False),
    )


def _scale_body(x_ref, d0_ref, d1_ref, xs_ref, dinv_ref, deginv_ref):
    deg = d0_ref[...] + d1_ref[...] + 1.0
    dinv = lax.rsqrt(deg)
    deginv = 1.0 / deg
    dinv_ref[...] = jnp.concatenate([dinv, dinv], axis=1)
    deginv_ref[...] = jnp.concatenate([deginv, deginv], axis=1)
    xs = x_ref[...] * dinv[:, :1]
    xs_ref[0] = xs[:, :128]
    xs_ref[1] = xs[:, 128:]


def _scale_call(x, d0, d1):
    return pl.pallas_call(
        _scale_body,
        grid=(GRID_TC,),
        in_specs=[
            pl.BlockSpec((ROW_TILE, F_IN), lambda i: (i, 0)),
            pl.BlockSpec((ROW_TILE, 8), lambda i: (i, 0)),
            pl.BlockSpec((ROW_TILE, 8), lambda i: (i, 0)),
        ],
        out_specs=[
            pl.BlockSpec((NC, ROW_TILE, 128), lambda i: (0, i, 0)),
            pl.BlockSpec((ROW_TILE, 16), lambda i: (i, 0)),
            pl.BlockSpec((ROW_TILE, 16), lambda i: (i, 0)),
        ],
        out_shape=[
            jax.ShapeDtypeStruct((NC, N, 128), jnp.float32),
            jax.ShapeDtypeStruct((N, 16), jnp.float32),
            jax.ShapeDtypeStruct((N, 16), jnp.float32),
        ],
    )(x, d0, d1)


def _dense_body(agg_ref, x_ref, dinv_ref, deginv_ref, w1_ref, b1_ref, w2_ref,
                z_ref, zs_ref):
    dinv = dinv_ref[:, :1]
    agg = jnp.concatenate([agg_ref[0], agg_ref[1]], axis=1)
    pre = dinv * agg + deginv_ref[:, :1] * x_ref[...]
    h = jnp.dot(pre, w1_ref[...], preferred_element_type=jnp.float32)
    h = jnp.maximum(h + b1_ref[...], 0.0)
    z = jnp.dot(h, w2_ref[...], preferred_element_type=jnp.float32)
    z_ref[...] = z
    zs = dinv * z
    zs_ref[0] = zs[:, :32]
    zs_ref[1] = zs[:, 32:]


def _dense_call(agg1, x, dinv, deginv, W1, b1, W2):
    return pl.pallas_call(
        _dense_body,
        grid=(GRID_TC,),
        in_specs=[
            pl.BlockSpec((NC, ROW_TILE, 128), lambda i: (0, i, 0)),
            pl.BlockSpec((ROW_TILE, F_IN), lambda i: (i, 0)),
            pl.BlockSpec((ROW_TILE, 16), lambda i: (i, 0)),
            pl.BlockSpec((ROW_TILE, 16), lambda i: (i, 0)),
            pl.BlockSpec((F_IN, HID), lambda i: (0, 0)),
            pl.BlockSpec((1, HID), lambda i: (0, 0)),
            pl.BlockSpec((HID, CLS), lambda i: (0, 0)),
        ],
        out_specs=[
            pl.BlockSpec((ROW_TILE, CLS), lambda i: (i, 0)),
            pl.BlockSpec((NC, ROW_TILE, 32), lambda i: (0, i, 0)),
        ],
        out_shape=[
            jax.ShapeDtypeStruct((N, CLS), jnp.float32),
            jax.ShapeDtypeStruct((NC, N, 32), jnp.float32),
        ],
    )(agg1, x, dinv, deginv, W1, b1, W2)


def _finish_body(agg_ref, z_ref, dinv_ref, deginv_ref, b2_ref, out_ref):
    agg = jnp.concatenate([agg_ref[0], agg_ref[1]], axis=1)
    out_ref[...] = (dinv_ref[:, :1] * agg + deginv_ref[:, :1] * z_ref[...]
                    + b2_ref[...])


def _finish_call(agg2, z, dinv, deginv, b2):
    return pl.pallas_call(
        _finish_body,
        grid=(GRID_TC,),
        in_specs=[
            pl.BlockSpec((NC, ROW_TILE, 32), lambda i: (0, i, 0)),
            pl.BlockSpec((ROW_TILE, CLS), lambda i: (i, 0)),
            pl.BlockSpec((ROW_TILE, 16), lambda i: (i, 0)),
            pl.BlockSpec((ROW_TILE, 16), lambda i: (i, 0)),
            pl.BlockSpec((1, CLS), lambda i: (0, 0)),
        ],
        out_specs=pl.BlockSpec((ROW_TILE, CLS), lambda i: (i, 0)),
        out_shape=jax.ShapeDtypeStruct((N, CLS), jnp.float32),
    )(agg2, z, dinv, deginv, b2)


def kernel(x, edge_index, W1, b1, W2, b2):
    src = edge_index[0]
    dst = edge_index[1]

    pad1 = EPAD - E
    src_p = jnp.concatenate([src, jnp.zeros((pad1,), jnp.int32)])
    src4 = jnp.stack([src_p, src_p + N]).reshape(NC, NS, CH_AGG, CHUNK)
    dst4 = jnp.concatenate(
        [dst, jnp.full((pad1,), N, jnp.int32)]).reshape(NS, CH_AGG, CHUNK)
    padd = EPAD_DEG - E
    dstd = jnp.concatenate(
        [dst, jnp.full((padd,), N, jnp.int32)]).reshape(NC, NS, CH_DEG, LANE)

    zeros128 = jnp.zeros((ZROWS, 128), jnp.float32)
    zeros32 = jnp.zeros((ZROWS, 32), jnp.float32)
    zeros8 = jnp.zeros((ZROWS, 8), jnp.float32)
    ones8 = jnp.ones((LANE, 8), jnp.float32)

    degp = _make_deg()(dstd, ones8, zeros8)

    xs2, dinv, deginv = _scale_call(x, degp[0], degp[1])

    agg1 = _make_agg(128, CH_AGG, 3)(src4, dst4, xs2.reshape(NC * N, 128),
                                     zeros128)

    z, zs2 = _dense_call(agg1, x, dinv, deginv, W1, b1.reshape(1, HID), W2)

    agg2 = _make_agg(32, CH_AGG, 6)(src4, dst4, zs2.reshape(NC * N, 32),
                                    zeros32)

    return _finish_call(agg2, z, dinv, deginv, b2.reshape(1, CLS))
